# baseline (speedup 1.0000x reference)
import functools

import jax
import jax.numpy as jnp
from jax import lax
from jax.experimental import pallas as pl
from jax.experimental.pallas import tpu as pltpu
from jax.experimental.pallas import tpu_sc as plsc


_NC = 2
_NS = 16
_ZR = 512


@functools.lru_cache(maxsize=None)
def _make(rows, cols, dtype_name):
    dtype = jnp.dtype(dtype_name)
    nw = _NC * _NS
    rpw = rows // nw
    nz = rpw // _ZR
    mesh = plsc.VectorSubcoreMesh(
        core_axis_name="c", subcore_axis_name="s",
        num_cores=_NC, num_subcores=_NS,
    )

    @functools.partial(
        pl.kernel,
        out_type=jax.ShapeDtypeStruct((2 * rows, cols), dtype),
        mesh=mesh,
        scratch_types=[
            pltpu.VMEM((_ZR, cols), dtype),
            pltpu.SemaphoreType.DMA,
            pltpu.SemaphoreType.DMA,
        ],
    )
    def sc_fill(x_hbm, o_hbm, zbuf, csem, zsem):
        wid = lax.axis_index("s") * _NC + lax.axis_index("c")
        base = wid * rpw

        cp = pltpu.make_async_copy(
            x_hbm.at[pl.ds(base, rpw), :],
            o_hbm.at[pl.ds(base, rpw), :],
            csem,
        )
        cp.start()

        zero16 = jnp.zeros((16,), dtype)

        def zrow(i, carry):
            zbuf[i, pl.ds(0, 16)] = zero16
            zbuf[i, pl.ds(16, 16)] = zero16
            return carry

        lax.fori_loop(0, _ZR, zrow, 0)

        zcopies = [
            pltpu.make_async_copy(
                zbuf,
                o_hbm.at[pl.ds(rows + base + k * _ZR, _ZR), :],
                zsem,
            )
            for k in range(nz)
        ]
        for zc in zcopies:
            zc.start()
        cp.wait()
        for zc in zcopies:
            zc.wait()

    return sc_fill


def kernel(psi):
    rows, cols = psi.shape
    return _make(rows, cols, psi.dtype.name)(psi)


# speedup vs baseline: 3.0679x; 3.0679x over previous
import functools

import jax
import jax.numpy as jnp
from jax import lax
from jax.experimental import pallas as pl
from jax.experimental.pallas import tpu as pltpu
from jax.experimental.pallas import tpu_sc as plsc


_NC = 2
_NS = 16
_ZR = 512


@functools.lru_cache(maxsize=None)
def _make(m, dtype_name):
    dtype = jnp.dtype(dtype_name)
    nw = _NC * _NS
    rpw = m // nw
    nz = rpw // _ZR
    mesh = plsc.VectorSubcoreMesh(
        core_axis_name="c", subcore_axis_name="s",
        num_cores=_NC, num_subcores=_NS,
    )

    @functools.partial(
        pl.kernel,
        out_type=jax.ShapeDtypeStruct((2 * m, 128), dtype),
        mesh=mesh,
        scratch_types=[
            pltpu.VMEM((_ZR, 128), dtype),
            pltpu.SemaphoreType.DMA,
            pltpu.SemaphoreType.DMA,
        ],
    )
    def sc_fill(x_hbm, o_hbm, zbuf, csem, zsem):
        wid = lax.axis_index("s") * _NC + lax.axis_index("c")
        base = wid * rpw

        cp = pltpu.make_async_copy(
            x_hbm.at[pl.ds(base, rpw), :],
            o_hbm.at[pl.ds(base, rpw), :],
            csem,
        )
        cp.start()

        zero16 = jnp.zeros((16,), dtype)

        def zrow(i, carry):
            for j in range(8):
                zbuf[i, pl.ds(16 * j, 16)] = zero16
            return carry

        lax.fori_loop(0, _ZR, zrow, 0)

        zcopies = [
            pltpu.make_async_copy(
                zbuf,
                o_hbm.at[pl.ds(m + base + k * _ZR, _ZR), :],
                zsem,
            )
            for k in range(nz)
        ]
        for zc in zcopies:
            zc.start()
        cp.wait()
        for zc in zcopies:
            zc.wait()

    return sc_fill


def kernel(psi):
    rows, cols = psi.shape
    m = (rows * cols) // 128
    flat = psi.reshape(m, 128)
    out = _make(m, psi.dtype.name)(flat)
    return out.reshape(2 * rows, cols)


# speedup vs baseline: 10.3760x; 3.3822x over previous
import jax
import jax.numpy as jnp
from jax.experimental import pallas as pl


_BLK = 16384


def _copy_zero_body(nb_in, x_ref, o_ref):
    i = pl.program_id(0)

    @pl.when(i < nb_in)
    def _copy():
        o_ref[...] = x_ref[...]

    @pl.when(i >= nb_in)
    def _zero():
        o_ref[...] = jnp.zeros_like(o_ref)


def kernel(psi):
    rows, cols = psi.shape
    m = (rows * cols) // 128
    flat = psi.reshape(m, 128)
    nb_in = m // _BLK

    out = pl.pallas_call(
        lambda x_ref, o_ref: _copy_zero_body(nb_in, x_ref, o_ref),
        grid=(2 * nb_in,),
        in_specs=[
            pl.BlockSpec((_BLK, 128), lambda i: (jnp.minimum(i, nb_in - 1), 0))
        ],
        out_specs=pl.BlockSpec((_BLK, 128), lambda i: (i, 0)),
        out_shape=jax.ShapeDtypeStruct((2 * m, 128), psi.dtype),
    )(flat)
    return out.reshape(2 * rows, cols)
